# baseline (speedup 1.0000x reference)
import functools

import jax
import jax.numpy as jnp
from jax import lax
from jax.experimental import pallas as pl
from jax.experimental.pallas import tpu as pltpu
from jax.experimental.pallas import tpu_sc as plsc

_N = 8192
_C = 8192
_D = 32
_TN = 512
_NW = 32
_BW = _N // _NW


def _vq_body(xm2_ref, e_ref, x2_ref, e2_ref, dist_ref, idx_ref):
    xm2 = xm2_ref[...]
    e = e_ref[...]
    x2 = x2_ref[...]
    e2 = e2_ref[...]
    xy = lax.dot_general(xm2, e, (((1,), (1,)), ((), ())),
                         preferred_element_type=jnp.float32)
    sq = (x2 + e2) + xy
    rt = jnp.where(sq > 0.0, sq * lax.rsqrt(sq), 0.0)
    dist_ref[...] = -rt
    m = jnp.min(rt, axis=1, keepdims=True)
    iota = lax.broadcasted_iota(jnp.int32, (_TN, _C), 1)
    idx_ref[...] = jnp.min(jnp.where(rt == m, iota, _C), axis=1, keepdims=True)


_sc_mesh = plsc.VectorSubcoreMesh(core_axis_name="c", subcore_axis_name="s")


@functools.partial(
    pl.kernel,
    mesh=_sc_mesh,
    out_type=jax.ShapeDtypeStruct((_N, 128), jnp.float32),
    scratch_types=[
        pltpu.VMEM((_BW,), jnp.int32),
        pltpu.VMEM((_BW, 128), jnp.float32),
        pltpu.SemaphoreType.DMA,
    ],
)
def _sc_gather(table_hbm, idx_hbm, out_hbm, idx_v, rows_v, sem):
    wid = lax.axis_index("s") * 2 + lax.axis_index("c")
    base = wid * _BW
    pltpu.sync_copy(idx_hbm.at[pl.ds(base, _BW)], idx_v)
    pltpu.async_copy(table_hbm.at[idx_v], rows_v, sem).wait()
    pltpu.sync_copy(rows_v, out_hbm.at[pl.ds(base, _BW)])


def kernel(x, embed):
    x = x.astype(jnp.float32)
    x2 = jnp.sum(x ** 2, axis=-1).reshape(_N, 1)
    e2 = jnp.sum(embed ** 2, axis=-1)
    xm2 = (x * -2.0)[0]
    dist, idx = pl.pallas_call(
        _vq_body,
        grid=(_N // _TN,),
        in_specs=[
            pl.BlockSpec((_TN, _D), lambda i: (i, 0)),
            pl.BlockSpec((_C, _D), lambda i: (0, 0)),
            pl.BlockSpec((_TN, 1), lambda i: (i, 0)),
            pl.BlockSpec((1, _C), lambda i: (0, 0)),
        ],
        out_specs=[
            pl.BlockSpec((_TN, _C), lambda i: (i, 0)),
            pl.BlockSpec((_TN, 1), lambda i: (i, 0)),
        ],
        out_shape=[
            jax.ShapeDtypeStruct((_N, _C), jnp.float32),
            jax.ShapeDtypeStruct((_N, 1), jnp.int32),
        ],
    )(xm2, embed[0], x2, e2)
    idx_flat = idx.reshape(_N)
    rows = _sc_gather(embed[0].reshape(_C // 4, 128), idx_flat >> 2)
    r4 = rows.reshape(_N, 4, _D)
    c = (idx_flat & 3)[:, None]
    q = jnp.where(c == 0, r4[:, 0],
                  jnp.where(c == 1, r4[:, 1],
                            jnp.where(c == 2, r4[:, 2], r4[:, 3])))
    return (q[None], idx_flat[None], dist[None])


# speedup vs baseline: 1.0131x; 1.0131x over previous
import functools

import jax
import jax.numpy as jnp
from jax import lax
from jax.experimental import pallas as pl
from jax.experimental.pallas import tpu as pltpu
from jax.experimental.pallas import tpu_sc as plsc

_N = 8192
_C = 8192
_D = 32
_TN = 512
_NW = 32
_BW = _N // _NW


def _vq_body(xm2_ref, e_ref, x2_ref, e2_ref, dist_ref, idx_ref):
    xm2 = xm2_ref[...]
    e = e_ref[...]
    x2 = x2_ref[...]
    e2 = e2_ref[...]
    xy = lax.dot_general(xm2, e, (((1,), (1,)), ((), ())),
                         preferred_element_type=jnp.float32)
    sq = (x2 + e2) + xy
    rt = jnp.where(sq > 0.0, sq * lax.rsqrt(sq), 0.0)
    dist_ref[...] = -rt
    m = jnp.min(rt, axis=1, keepdims=True)
    iota = lax.broadcasted_iota(jnp.int32, (_TN, _C), 1)
    idx_ref[...] = jnp.min(jnp.where(rt == m, iota, _C), axis=1, keepdims=True)


_sc_mesh = plsc.VectorSubcoreMesh(core_axis_name="c", subcore_axis_name="s")


@functools.partial(
    pl.kernel,
    mesh=_sc_mesh,
    out_type=jax.ShapeDtypeStruct((_N, 128), jnp.float32),
    scratch_types=[
        pltpu.VMEM((_BW,), jnp.int32),
        pltpu.VMEM((_BW, 128), jnp.float32),
        pltpu.SemaphoreType.DMA,
    ],
)
def _sc_gather(table_hbm, idx_hbm, out_hbm, idx_v, rows_v, sem):
    wid = lax.axis_index("s") * 2 + lax.axis_index("c")
    base = wid * _BW
    pltpu.sync_copy(idx_hbm.at[pl.ds(base, _BW)], idx_v)
    pltpu.async_copy(table_hbm.at[idx_v], rows_v, sem).wait()
    pltpu.sync_copy(rows_v, out_hbm.at[pl.ds(base, _BW)])


def kernel(x, embed):
    x = x.astype(jnp.float32)
    x2 = jnp.sum(x ** 2, axis=-1).reshape(_N, 1)
    e2 = jnp.sum(embed ** 2, axis=-1)
    xm2 = (x * -2.0)[0]
    dist, idx = pl.pallas_call(
        _vq_body,
        grid=(_N // _TN,),
        in_specs=[
            pl.BlockSpec((_TN, _D), lambda i: (i, 0)),
            pl.BlockSpec((_C, _D), lambda i: (0, 0)),
            pl.BlockSpec((_TN, 1), lambda i: (i, 0)),
            pl.BlockSpec((1, _C), lambda i: (0, 0)),
        ],
        out_specs=[
            pl.BlockSpec((_TN, _C), lambda i: (i, 0)),
            pl.BlockSpec((_TN, 1), lambda i: (i, 0)),
        ],
        out_shape=[
            jax.ShapeDtypeStruct((_N, _C), jnp.float32),
            jax.ShapeDtypeStruct((_N, 1), jnp.int32),
        ],
    )(xm2, embed[0], x2, e2)
    idx_flat = idx.reshape(_N)
    e_pad = jnp.pad(embed[0], ((0, 0), (0, 128 - _D)))
    q = _sc_gather(e_pad, idx_flat)[:, :_D]
    return (q[None], idx_flat[None], dist[None])
